# baseline (speedup 1.0000x reference)
import functools

import jax
import jax.numpy as jnp
from jax import lax
from jax.experimental import pallas as pl
from jax.experimental.pallas import tpu as pltpu
from jax.experimental.pallas import tpu_sc as plsc

B, HW, D, NPQ = 16, 1024, 96, 256
NQ, NP = 8, 16
NROWS = B * NQ * NP
NW = 32
ROWS_PER_W = NROWS // NW


def _attn_topk_body(sz_ref, zp_ref, idx_ref):
    b = pl.program_id(0)
    a = sz_ref[0]
    zp = zp_ref[0]
    attn = lax.dot_general(a, zp, (((1,), (1,)), ((), ())),
                           preferred_element_type=jnp.float32)
    iota = lax.broadcasted_iota(jnp.int32, (NQ, HW), 1)
    sels = []
    for _ in range(NP):
        m = jnp.max(attn, axis=1, keepdims=True)
        cand = jnp.where(attn == m, iota, HW)
        sel = jnp.min(cand, axis=1, keepdims=True)
        sels.append(sel)
        attn = jnp.where(iota == sel, -jnp.inf, attn)
    r = jnp.concatenate(sels, axis=1)
    idx_ref[0] = r + b * HW


def _attn_topk(sample_z, zp):
    return pl.pallas_call(
        _attn_topk_body,
        grid=(B,),
        in_specs=[
            pl.BlockSpec((1, NQ, D), lambda b: (b, 0, 0)),
            pl.BlockSpec((1, HW, D), lambda b: (b, 0, 0)),
        ],
        out_specs=pl.BlockSpec((1, NQ, NP), lambda b: (b, 0, 0)),
        out_shape=jax.ShapeDtypeStruct((B, NQ, NP), jnp.int32),
    )(sample_z, zp)


def _sc_gather(table, idx):
    @functools.partial(
        pl.kernel,
        mesh=plsc.VectorSubcoreMesh(core_axis_name="c", subcore_axis_name="s"),
        out_type=jax.ShapeDtypeStruct((NROWS, NPQ), jnp.float32),
        scratch_types=[
            pltpu.VMEM((ROWS_PER_W,), jnp.int32),
            pltpu.VMEM((ROWS_PER_W, NPQ), jnp.float32),
            pltpu.SemaphoreType.DMA,
        ],
    )
    def gather_kernel(table_hbm, idx_hbm, out_hbm, idx_v, rows_v, sem):
        wid = lax.axis_index("s") * 2 + lax.axis_index("c")
        base = wid * ROWS_PER_W
        pltpu.sync_copy(idx_hbm.at[pl.ds(base, ROWS_PER_W)], idx_v)
        pltpu.async_copy(table_hbm.at[idx_v], rows_v, sem).wait()
        pltpu.sync_copy(rows_v, out_hbm.at[pl.ds(base, ROWS_PER_W)])

    return gather_kernel(table, idx)


def _jsd_body(p_ref, g_ref, out_ref):
    p8 = p_ref[0]
    q = g_ref[0]
    p = jnp.concatenate([p8] * NP, axis=0)
    m = jnp.log(jnp.clip((p + q) * 0.5, 1e-7, 1.0))
    tp = jnp.where(p > 0, p * (jnp.log(jnp.where(p > 0, p, 1.0)) - m), 0.0)
    tq = jnp.where(q > 0, q * (jnp.log(jnp.where(q > 0, q, 1.0)) - m), 0.0)
    out_ref[0] = jnp.sum(tp + tq)[None, None]


def _jsd(sample_z_dis, gathered):
    return pl.pallas_call(
        _jsd_body,
        grid=(B,),
        in_specs=[
            pl.BlockSpec((1, NQ, NPQ), lambda b: (b, 0, 0)),
            pl.BlockSpec((1, NQ * NP, NPQ), lambda b: (b, 0, 0)),
        ],
        out_specs=pl.BlockSpec((1, 1, 1), lambda b: (b, 0, 0)),
        out_shape=jax.ShapeDtypeStruct((B, 1, 1), jnp.float32),
    )(sample_z_dis, gathered)


def kernel(z, z_pos, z_dis, z_pos_dis):
    rkey = jax.random.key(42)
    rand_11 = jax.random.randint(rkey, (B, NQ), 0, HW)
    gidx = rand_11 + (jnp.arange(B) * HW)[:, None]
    sample_z = jnp.take(z.reshape(B * HW, D), gidx, axis=0)
    sample_z_dis = jnp.take(z_dis.reshape(B * HW, NPQ), gidx, axis=0)

    idx = _attn_topk(sample_z, z_pos.reshape(B, HW, D))
    gathered = _sc_gather(z_pos_dis.reshape(B * HW, NPQ),
                          idx.reshape(NROWS))
    partial = _jsd(sample_z_dis, gathered.reshape(B, NQ * NP, NPQ))
    return jnp.sum(partial) * jnp.float32(0.5 / NROWS)


# speedup vs baseline: 1.9945x; 1.9945x over previous
import functools

import jax
import jax.numpy as jnp
from jax import lax
from jax.experimental import pallas as pl
from jax.experimental.pallas import tpu as pltpu
from jax.experimental.pallas import tpu_sc as plsc

B, HW, D, NPQ = 16, 1024, 96, 256
NQ, NP = 8, 16
NROWS = B * NQ * NP
NW = 32
ROWS_PER_W = NROWS // NW


def _attn_topk_body(sz_ref, zp_ref, idx_ref):
    attn = jnp.concatenate(
        [lax.dot_general(sz_ref[b], zp_ref[b], (((1,), (1,)), ((), ())),
                         preferred_element_type=jnp.float32)
         for b in range(B)], axis=0)
    iota = lax.broadcasted_iota(jnp.int32, (B * NQ, HW), 1)
    sels = []
    for _ in range(NP):
        m = jnp.max(attn, axis=1, keepdims=True)
        cand = jnp.where(attn == m, iota, HW)
        sel = jnp.min(cand, axis=1, keepdims=True)
        sels.append(sel)
        attn = jnp.where(iota == sel, -jnp.inf, attn)
    r = jnp.concatenate(sels, axis=1)
    base = lax.broadcasted_iota(jnp.int32, (B * NQ, NP), 0) // NQ * HW
    idx_ref[...] = (r + base).reshape(B, NQ, NP)


def _attn_topk(sample_z, zp):
    return pl.pallas_call(
        _attn_topk_body,
        out_shape=jax.ShapeDtypeStruct((B, NQ, NP), jnp.int32),
    )(sample_z, zp)


def _sc_gather(table, idx):
    @functools.partial(
        pl.kernel,
        mesh=plsc.VectorSubcoreMesh(core_axis_name="c", subcore_axis_name="s"),
        out_type=jax.ShapeDtypeStruct((NROWS, NPQ), jnp.float32),
        scratch_types=[
            pltpu.VMEM((ROWS_PER_W,), jnp.int32),
            pltpu.VMEM((ROWS_PER_W, NPQ), jnp.float32),
            pltpu.SemaphoreType.DMA,
        ],
    )
    def gather_kernel(table_hbm, idx_hbm, out_hbm, idx_v, rows_v, sem):
        wid = lax.axis_index("s") * 2 + lax.axis_index("c")
        base = wid * ROWS_PER_W
        pltpu.sync_copy(idx_hbm.at[pl.ds(base, ROWS_PER_W)], idx_v)
        pltpu.async_copy(table_hbm.at[idx_v], rows_v, sem).wait()
        pltpu.sync_copy(rows_v, out_hbm.at[pl.ds(base, ROWS_PER_W)])

    return gather_kernel(table, idx)


def _jsd_body(p_ref, g_ref, out_ref):
    p8 = p_ref[0]
    q = g_ref[0]
    p = jnp.concatenate([p8] * NP, axis=0)
    m = jnp.log(jnp.clip((p + q) * 0.5, 1e-7, 1.0))
    tp = jnp.where(p > 0, p * (jnp.log(jnp.where(p > 0, p, 1.0)) - m), 0.0)
    tq = jnp.where(q > 0, q * (jnp.log(jnp.where(q > 0, q, 1.0)) - m), 0.0)
    out_ref[0] = jnp.sum(tp + tq)[None, None]


def _jsd(sample_z_dis, gathered):
    return pl.pallas_call(
        _jsd_body,
        grid=(B,),
        in_specs=[
            pl.BlockSpec((1, NQ, NPQ), lambda b: (b, 0, 0)),
            pl.BlockSpec((1, NQ * NP, NPQ), lambda b: (b, 0, 0)),
        ],
        out_specs=pl.BlockSpec((1, 1, 1), lambda b: (b, 0, 0)),
        out_shape=jax.ShapeDtypeStruct((B, 1, 1), jnp.float32),
    )(sample_z_dis, gathered)


def kernel(z, z_pos, z_dis, z_pos_dis):
    rkey = jax.random.key(42)
    rand_11 = jax.random.randint(rkey, (B, NQ), 0, HW)
    gidx = rand_11 + (jnp.arange(B) * HW)[:, None]
    sample_z = jnp.take(z.reshape(B * HW, D), gidx, axis=0)
    sample_z_dis = jnp.take(z_dis.reshape(B * HW, NPQ), gidx, axis=0)

    idx = _attn_topk(sample_z, z_pos.reshape(B, HW, D))
    gathered = _sc_gather(z_pos_dis.reshape(B * HW, NPQ),
                          idx.reshape(NROWS))
    partial = _jsd(sample_z_dis, gathered.reshape(B, NQ * NP, NPQ))
    return jnp.sum(partial) * jnp.float32(0.5 / NROWS)
